# baseline (speedup 1.0000x reference)
import functools

import jax
import jax.numpy as jnp
from jax import lax
from jax.experimental import pallas as pl
from jax.experimental.pallas import tpu as pltpu
from jax.experimental.pallas import tpu_sc as plsc

N = 10000
D = 128
E = 320000
NP = 10240
ROWS = NP // 128
NC = 2
NS = 16
NW = NC * NS
K = 128
CH = 80
EW = K * CH
EP = EW * NW
IDXC = 1280
MB = 1280
GRID = NP // MB

_mesh = plsc.VectorSubcoreMesh(
    core_axis_name="c", subcore_axis_name="s", num_cores=NC, num_subcores=NS
)


def _zero_vmem_rows(ref, nrows):
    zero16 = jnp.zeros((16,), jnp.float32)

    def zrow(r, carry):
        for j in range(8):
            ref[r, pl.ds(j * 16, 16)] = zero16
        return carry

    lax.fori_loop(0, nrows, zrow, 0)


@functools.partial(
    pl.kernel,
    out_type=jax.ShapeDtypeStruct((NC, NP), jnp.float32),
    mesh=_mesh,
    scratch_types=[
        pltpu.VMEM((NP,), jnp.float32),
        pltpu.VMEM((IDXC,), jnp.int32),
        pltpu.VMEM((NP // NS,), jnp.float32),
        pltpu.VMEM((NP // NS,), jnp.float32),
        pltpu.VMEM_SHARED((NS, NP), jnp.float32),
    ],
    compiler_params=pltpu.CompilerParams(needs_layout_passes=False),
)
def _sc_deg(dst_hbm, out_hbm, hist, idxb, racc, rtmp, stage_sh):
    c = lax.axis_index("c")
    s = lax.axis_index("s")
    w = s * NC + c
    seg = NP // NS

    zero16 = jnp.zeros((16,), jnp.float32)

    def zel(i, carry):
        hist[pl.ds(pl.multiple_of(i * 16, 16), 16)] = zero16
        return carry

    lax.fori_loop(0, NP // 16, zel, 0)

    base = w * EW
    ones16 = jnp.ones((16,), jnp.float32)

    def outer(k, carry):
        off = pl.multiple_of(base + k * IDXC, 8)
        pltpu.sync_copy(dst_hbm.at[pl.ds(off, IDXC)], idxb)

        def inner(i, carry2):
            v = idxb[pl.ds(pl.multiple_of(i * 16, 16), 16)]
            plsc.addupdate_scatter(hist, [v], ones16)
            return carry2

        lax.fori_loop(0, IDXC // 16, inner, 0)
        return carry

    lax.fori_loop(0, EW // IDXC, outer, 0)

    pltpu.sync_copy(hist, stage_sh.at[s])
    plsc.subcore_barrier()

    cbase = pl.multiple_of(s * seg, 8)
    pltpu.sync_copy(stage_sh.at[0, pl.ds(cbase, seg)], racc)
    for k in range(1, NS):
        pltpu.sync_copy(stage_sh.at[k, pl.ds(cbase, seg)], rtmp)

        def addel(i, carry):
            sl = pl.ds(pl.multiple_of(i * 16, 16), 16)
            racc[sl] = racc[sl] + rtmp[sl]
            return carry

        lax.fori_loop(0, seg // 16, addel, 0)
    pltpu.sync_copy(racc, out_hbm.at[c, pl.ds(cbase, seg)])


@functools.partial(
    pl.kernel,
    out_type=jax.ShapeDtypeStruct((NC, NP, D), jnp.float32),
    mesh=_mesh,
    scratch_types=[
        pltpu.VMEM((K, D), jnp.float32),
        pltpu.VMEM((K, D), jnp.float32),
        pltpu.VMEM((4, K), jnp.int32),
        pltpu.VMEM((4, K), jnp.int32),
        pltpu.VMEM_SHARED((NP, D), jnp.float32),
        pltpu.SemaphoreType.DMA,
        pltpu.SemaphoreType.DMA,
        pltpu.SemaphoreType.DMA,
        pltpu.SemaphoreType.DMA,
        pltpu.SemaphoreType.DMA,
        pltpu.SemaphoreType.DMA,
    ],
    compiler_params=pltpu.CompilerParams(needs_layout_passes=False),
)
def _sc_seg(x_hbm, src_hbm, dst_hbm, out_hbm, rows_a, rows_b, srci, dsti,
            acc_sh, sem_a, sem_b, si0, si1, si2, si3):
    c = lax.axis_index("c")
    s = lax.axis_index("s")
    w = s * NC + c
    base = w * EW
    rows = (rows_a, rows_b)
    rsem = (sem_a, sem_b)
    isem = (si0, si1, si2, si3)

    def _ioff(j):
        return pl.multiple_of(jnp.minimum(base + j * K, EP - K), 8)

    def issue_idx(j, q):
        off = _ioff(j)
        pltpu.async_copy(src_hbm.at[pl.ds(off, K)], srci.at[q], isem[q])
        pltpu.async_copy(dst_hbm.at[pl.ds(off, K)], dsti.at[q], isem[q])

    def wait_idx(j, q):
        off = _ioff(j)
        pltpu.make_async_copy(src_hbm.at[pl.ds(off, K)], srci.at[q], isem[q]).wait()
        pltpu.make_async_copy(dst_hbm.at[pl.ds(off, K)], dsti.at[q], isem[q]).wait()

    _zero_vmem_rows(rows_a, K)
    srows = NP // NS
    for k in range(srows // K):
        off = pl.multiple_of(s * srows + k * K, 8)
        pltpu.sync_copy(rows_a, acc_sh.at[pl.ds(off, K)])
    plsc.subcore_barrier()

    for q in range(4):
        issue_idx(q, q)
    wait_idx(0, 0)
    wait_idx(1, 1)
    pltpu.async_copy(x_hbm.at[srci.at[0]], rows_a, sem_a)
    pltpu.async_copy(x_hbm.at[srci.at[1]], rows_b, sem_b)

    def step(j2, carry):
        jbase = pl.multiple_of(j2 * 4, 4)
        for b in range(4):
            j = jbase + b
            rb = rows[b % 2]
            rs = rsem[b % 2]
            qn = (b + 2) % 4
            wait_idx(j + 2, qn)
            pltpu.make_async_copy(x_hbm.at[srci.at[b]], rb, rs).wait()
            pltpu.sync_copy(rb, acc_sh.at[dsti.at[b]], add=True)
            pltpu.async_copy(x_hbm.at[srci.at[qn]], rb, rs)
            issue_idx(j + 4, b)
        return carry

    lax.fori_loop(0, CH // 4, step, 0)
    pltpu.make_async_copy(x_hbm.at[srci.at[0]], rows_a, sem_a).wait()
    pltpu.make_async_copy(x_hbm.at[srci.at[1]], rows_b, sem_b).wait()
    wait_idx(CH + 2, 2)
    wait_idx(CH + 3, 3)
    plsc.subcore_barrier()

    for k in range(srows // K):
        off = pl.multiple_of(s * srows + k * K, 8)
        pltpu.sync_copy(acc_sh.at[pl.ds(off, K)], out_hbm.at[c, pl.ds(off, K)])


def _tc_in_body(x_ref, w_ref, b_ref, nc_ref, h_ref, xs_ref):
    h = jnp.dot(x_ref[...], w_ref[...], preferred_element_type=jnp.float32)
    h = h + b_ref[...]
    h_ref[...] = h
    xs_ref[...] = h * nc_ref[...]


def _tc_in(x, w, b, normc):
    return pl.pallas_call(
        _tc_in_body,
        grid=(GRID,),
        in_specs=[
            pl.BlockSpec((MB, D), lambda i: (i, 0)),
            pl.BlockSpec((D, D), lambda i: (0, 0)),
            pl.BlockSpec((1, D), lambda i: (0, 0)),
            pl.BlockSpec((MB, D), lambda i: (i, 0)),
        ],
        out_specs=[pl.BlockSpec((MB, D), lambda i: (i, 0))] * 2,
        out_shape=[jax.ShapeDtypeStruct((NP, D), jnp.float32)] * 2,
    )(x, w, b, normc)


def _tc_mid_body(p0_ref, p1_ref, nc_ref, w_ref, b_ref, h0_ref, x1_ref):
    y = (p0_ref[...] + p1_ref[...]) * nc_ref[...]
    t = jnp.dot(y, w_ref[...], preferred_element_type=jnp.float32) + b_ref[...]
    t = jnp.maximum(t, 0.0) + h0_ref[...]
    x1_ref[...] = t * nc_ref[...]


def _tc_mid(p0, p1, normc, w, b, h0):
    return pl.pallas_call(
        _tc_mid_body,
        grid=(GRID,),
        in_specs=[
            pl.BlockSpec((MB, D), lambda i: (i, 0)),
            pl.BlockSpec((MB, D), lambda i: (i, 0)),
            pl.BlockSpec((MB, D), lambda i: (i, 0)),
            pl.BlockSpec((D, D), lambda i: (0, 0)),
            pl.BlockSpec((1, D), lambda i: (0, 0)),
            pl.BlockSpec((MB, D), lambda i: (i, 0)),
        ],
        out_specs=pl.BlockSpec((MB, D), lambda i: (i, 0)),
        out_shape=jax.ShapeDtypeStruct((NP, D), jnp.float32),
    )(p0, p1, normc, w, b, h0)


def _tc_out_body(p0_ref, p1_ref, nc_ref, w1_ref, b1_ref, wo_ref, bo_ref, o_ref):
    y = (p0_ref[...] + p1_ref[...]) * nc_ref[...]
    h2 = jnp.dot(y, w1_ref[...], preferred_element_type=jnp.float32) + b1_ref[...]
    h2 = jnp.maximum(h2, 0.0)
    o_ref[...] = (
        jnp.dot(h2, wo_ref[...], preferred_element_type=jnp.float32) + bo_ref[...]
    )


def _tc_out(p0, p1, normc, w1, b1, wo, bo):
    return pl.pallas_call(
        _tc_out_body,
        grid=(GRID,),
        in_specs=[
            pl.BlockSpec((MB, D), lambda i: (i, 0)),
            pl.BlockSpec((MB, D), lambda i: (i, 0)),
            pl.BlockSpec((MB, D), lambda i: (i, 0)),
            pl.BlockSpec((D, D), lambda i: (0, 0)),
            pl.BlockSpec((1, D), lambda i: (0, 0)),
            pl.BlockSpec((D, D), lambda i: (0, 0)),
            pl.BlockSpec((1, D), lambda i: (0, 0)),
        ],
        out_specs=pl.BlockSpec((MB, D), lambda i: (i, 0)),
        out_shape=jax.ShapeDtypeStruct((NP, D), jnp.float32),
    )(p0, p1, normc, w1, b1, wo, bo)


def kernel(features, edge_index, W_in, b_in, W0, b0, W1, b1, W_out, b_out):
    src = edge_index[0].astype(jnp.int32)
    dst = edge_index[1].astype(jnp.int32)
    pad = EP - E
    srcp = jnp.concatenate([src, jnp.zeros((pad,), jnp.int32)])
    dstp = jnp.concatenate([dst, jnp.full((pad,), NP - 8, jnp.int32)])
    xp = jnp.pad(features, ((0, NP - N), (0, 0)))

    degp = _sc_deg(dstp)
    deg = degp[0] + degp[1]
    norm = lax.rsqrt(jnp.maximum(deg, 1.0))
    normc = jnp.broadcast_to(norm[:, None], (NP, D))

    h0, x0 = _tc_in(xp, W_in, b_in.reshape(1, D), normc)
    p = _sc_seg(x0, srcp, dstp)
    x1 = _tc_mid(p[0], p[1], normc, W0, b0.reshape(1, D), h0)
    p = _sc_seg(x1, srcp, dstp)
    out = _tc_out(p[0], p[1], normc, W1, b1.reshape(1, D), W_out, b_out.reshape(1, D))
    return out[:N]


# speedup vs baseline: 1.5859x; 1.5859x over previous
import functools

import jax
import jax.numpy as jnp
from jax import lax
from jax.experimental import pallas as pl
from jax.experimental.pallas import tpu as pltpu
from jax.experimental.pallas import tpu_sc as plsc

N = 10000
D = 128
E = 320000
NP = 10240
ROWS = NP // 128
NC = 2
NS = 16
NW = NC * NS
K = 128
CH = 80
EW = K * CH
EP = EW * NW
IDXC = 1280
MB = 1280
GRID = NP // MB

_mesh = plsc.VectorSubcoreMesh(
    core_axis_name="c", subcore_axis_name="s", num_cores=NC, num_subcores=NS
)


def _zero_vmem_rows(ref, nrows, ncols=128):
    zero16 = jnp.zeros((16,), jnp.float32)

    def zrow(r, carry):
        for j in range(ncols // 16):
            ref[r, pl.ds(j * 16, 16)] = zero16
        return carry

    lax.fori_loop(0, nrows, zrow, 0)


@functools.partial(
    pl.kernel,
    out_type=jax.ShapeDtypeStruct((NC, NP), jnp.float32),
    mesh=_mesh,
    scratch_types=[
        pltpu.VMEM((NP,), jnp.float32),
        pltpu.VMEM((IDXC,), jnp.int32),
        pltpu.VMEM((NP // NS,), jnp.float32),
        pltpu.VMEM((NP // NS,), jnp.float32),
        pltpu.VMEM_SHARED((NS, NP), jnp.float32),
    ],
    compiler_params=pltpu.CompilerParams(needs_layout_passes=False),
)
def _sc_deg(dst_hbm, out_hbm, hist, idxb, racc, rtmp, stage_sh):
    c = lax.axis_index("c")
    s = lax.axis_index("s")
    w = s * NC + c
    seg = NP // NS

    zero16 = jnp.zeros((16,), jnp.float32)

    def zel(i, carry):
        hist[pl.ds(pl.multiple_of(i * 16, 16), 16)] = zero16
        return carry

    lax.fori_loop(0, NP // 16, zel, 0)

    base = w * EW
    ones16 = jnp.ones((16,), jnp.float32)

    def outer(k, carry):
        off = pl.multiple_of(base + k * IDXC, 8)
        pltpu.sync_copy(dst_hbm.at[pl.ds(off, IDXC)], idxb)

        def inner(i, carry2):
            v = idxb[pl.ds(pl.multiple_of(i * 16, 16), 16)]
            plsc.addupdate_scatter(hist, [v], ones16)
            return carry2

        lax.fori_loop(0, IDXC // 16, inner, 0)
        return carry

    lax.fori_loop(0, EW // IDXC, outer, 0)

    pltpu.sync_copy(hist, stage_sh.at[s])
    plsc.subcore_barrier()

    cbase = pl.multiple_of(s * seg, 8)
    pltpu.sync_copy(stage_sh.at[0, pl.ds(cbase, seg)], racc)
    for k in range(1, NS):
        pltpu.sync_copy(stage_sh.at[k, pl.ds(cbase, seg)], rtmp)

        def addel(i, carry):
            sl = pl.ds(pl.multiple_of(i * 16, 16), 16)
            racc[sl] = racc[sl] + rtmp[sl]
            return carry

        lax.fori_loop(0, seg // 16, addel, 0)
    pltpu.sync_copy(racc, out_hbm.at[c, pl.ds(cbase, seg)])


HD = D // 2


@functools.partial(
    pl.kernel,
    out_type=jax.ShapeDtypeStruct((NC, 2, NP, HD), jnp.float32),
    mesh=_mesh,
    scratch_types=[
        pltpu.VMEM((K, HD), jnp.float32),
        pltpu.VMEM((CH, K), jnp.int32),
        pltpu.VMEM((CH, K), jnp.int32),
        pltpu.VMEM_SHARED((NP, HD), jnp.float32),
        pltpu.VMEM_SHARED((NP, HD), jnp.float32),
        pltpu.SemaphoreType.DMA,
    ],
    compiler_params=pltpu.CompilerParams(
        needs_layout_passes=False, use_tc_tiling_on_sc=False
    ),
)
def _sc_seg(x_hbm, src_hbm, dst_hbm, out_hbm, rows, srcb, dstb,
            x_sh, acc_sh, sem):
    c = lax.axis_index("c")
    s = lax.axis_index("s")
    w = s * NC + c
    srows = NP // NS

    pltpu.sync_copy(src_hbm.at[w], srcb)
    pltpu.sync_copy(dst_hbm.at[w], dstb)

    for h in range(2):
        rbase = pl.multiple_of(s * srows, 8)
        pltpu.sync_copy(
            x_hbm.at[h, pl.ds(rbase, srows)],
            x_sh.at[pl.ds(rbase, srows)],
        )
        _zero_vmem_rows(rows, K, HD)
        for k in range(srows // K):
            off = pl.multiple_of(s * srows + k * K, 8)
            pltpu.sync_copy(rows, acc_sh.at[pl.ds(off, K)])
        plsc.subcore_barrier()

        def chunk(j, carry):
            pltpu.async_copy(x_sh.at[srcb.at[j]], rows, sem).wait()
            pltpu.sync_copy(rows, acc_sh.at[dstb.at[j]], add=True)
            return carry

        lax.fori_loop(0, CH, chunk, 0)
        plsc.subcore_barrier()

        for k in range(srows // K):
            off = pl.multiple_of(s * srows + k * K, 8)
            pltpu.sync_copy(acc_sh.at[pl.ds(off, K)], out_hbm.at[c, h, pl.ds(off, K)])
        plsc.subcore_barrier()


def _tc_in_body(x_ref, w_ref, b_ref, nc_ref, h_ref, xs_ref):
    h = jnp.dot(x_ref[...], w_ref[...], preferred_element_type=jnp.float32)
    h = h + b_ref[...]
    h_ref[...] = h
    xs_ref[...] = h * nc_ref[...]


def _tc_in(x, w, b, normc):
    return pl.pallas_call(
        _tc_in_body,
        grid=(GRID,),
        in_specs=[
            pl.BlockSpec((MB, D), lambda i: (i, 0)),
            pl.BlockSpec((D, D), lambda i: (0, 0)),
            pl.BlockSpec((1, D), lambda i: (0, 0)),
            pl.BlockSpec((MB, D), lambda i: (i, 0)),
        ],
        out_specs=[pl.BlockSpec((MB, D), lambda i: (i, 0))] * 2,
        out_shape=[jax.ShapeDtypeStruct((NP, D), jnp.float32)] * 2,
    )(x, w, b, normc)


def _tc_mid_body(p_ref, nc_ref, w_ref, b_ref, h0_ref, x1_ref):
    y = p_ref[...] * nc_ref[...]
    t = jnp.dot(y, w_ref[...], preferred_element_type=jnp.float32) + b_ref[...]
    t = jnp.maximum(t, 0.0) + h0_ref[...]
    x1_ref[...] = t * nc_ref[...]


def _tc_mid(p, normc, w, b, h0):
    return pl.pallas_call(
        _tc_mid_body,
        grid=(GRID,),
        in_specs=[
            pl.BlockSpec((MB, D), lambda i: (i, 0)),
            pl.BlockSpec((MB, D), lambda i: (i, 0)),
            pl.BlockSpec((D, D), lambda i: (0, 0)),
            pl.BlockSpec((1, D), lambda i: (0, 0)),
            pl.BlockSpec((MB, D), lambda i: (i, 0)),
        ],
        out_specs=pl.BlockSpec((MB, D), lambda i: (i, 0)),
        out_shape=jax.ShapeDtypeStruct((NP, D), jnp.float32),
    )(p, normc, w, b, h0)


def _tc_out_body(p_ref, nc_ref, w1_ref, b1_ref, wo_ref, bo_ref, o_ref):
    y = p_ref[...] * nc_ref[...]
    h2 = jnp.dot(y, w1_ref[...], preferred_element_type=jnp.float32) + b1_ref[...]
    h2 = jnp.maximum(h2, 0.0)
    o_ref[...] = (
        jnp.dot(h2, wo_ref[...], preferred_element_type=jnp.float32) + bo_ref[...]
    )


def _tc_out(p, normc, w1, b1, wo, bo):
    return pl.pallas_call(
        _tc_out_body,
        grid=(GRID,),
        in_specs=[
            pl.BlockSpec((MB, D), lambda i: (i, 0)),
            pl.BlockSpec((MB, D), lambda i: (i, 0)),
            pl.BlockSpec((D, D), lambda i: (0, 0)),
            pl.BlockSpec((1, D), lambda i: (0, 0)),
            pl.BlockSpec((D, D), lambda i: (0, 0)),
            pl.BlockSpec((1, D), lambda i: (0, 0)),
        ],
        out_specs=pl.BlockSpec((MB, D), lambda i: (i, 0)),
        out_shape=jax.ShapeDtypeStruct((NP, D), jnp.float32),
    )(p, normc, w1, b1, wo, bo)


def kernel(features, edge_index, W_in, b_in, W0, b0, W1, b1, W_out, b_out):
    src = edge_index[0].astype(jnp.int32)
    dst = edge_index[1].astype(jnp.int32)
    pad = EP - E
    srcp = jnp.concatenate([src, jnp.zeros((pad,), jnp.int32)])
    dstp = jnp.concatenate([dst, jnp.full((pad,), NP - 8, jnp.int32)])
    src3 = srcp.reshape(NW, CH, K)
    dst3 = dstp.reshape(NW, CH, K)
    xp = jnp.pad(features, ((0, NP - N), (0, 0)))

    degp = _sc_deg(dstp)
    deg = degp[0] + degp[1]
    norm = lax.rsqrt(jnp.maximum(deg, 1.0))
    normc = jnp.broadcast_to(norm[:, None], (NP, D))

    def _seg(x):
        xh = x.reshape(NP, 2, HD).transpose(1, 0, 2)
        p = _sc_seg(xh, src3, dst3)
        ps = p[0] + p[1]
        return jnp.concatenate([ps[0], ps[1]], axis=1)

    h0, x0 = _tc_in(xp, W_in, b_in.reshape(1, D), normc)
    x1 = _tc_mid(_seg(x0), normc, W0, b0.reshape(1, D), h0)
    out = _tc_out(_seg(x1), normc, W1, b1.reshape(1, D), W_out, b_out.reshape(1, D))
    return out[:N]


# speedup vs baseline: 1.6404x; 1.0344x over previous
import functools

import jax
import jax.numpy as jnp
from jax import lax
from jax.experimental import pallas as pl
from jax.experimental.pallas import tpu as pltpu
from jax.experimental.pallas import tpu_sc as plsc

N = 10000
D = 128
E = 320000
NP = 10240
ROWS = NP // 128
NC = 2
NS = 16
NW = NC * NS
K = 128
CH = 80
EW = K * CH
EP = EW * NW
IDXC = 1280
MB = 400
GRID = N // MB

_mesh = plsc.VectorSubcoreMesh(
    core_axis_name="c", subcore_axis_name="s", num_cores=NC, num_subcores=NS
)


def _zero_vmem_rows(ref, nrows, ncols=128):
    zero16 = jnp.zeros((16,), jnp.float32)

    def zrow(r, carry):
        for j in range(ncols // 16):
            ref[r, pl.ds(j * 16, 16)] = zero16
        return carry

    lax.fori_loop(0, nrows, zrow, 0)


@functools.partial(
    pl.kernel,
    out_type=jax.ShapeDtypeStruct((NC, NP), jnp.float32),
    mesh=_mesh,
    scratch_types=[
        pltpu.VMEM((NP,), jnp.float32),
        pltpu.VMEM((IDXC,), jnp.int32),
        pltpu.VMEM((NP // NS,), jnp.float32),
        pltpu.VMEM((NP // NS,), jnp.float32),
        pltpu.VMEM_SHARED((NS, NP), jnp.float32),
    ],
    compiler_params=pltpu.CompilerParams(needs_layout_passes=False),
)
def _sc_deg(dst_hbm, out_hbm, hist, idxb, racc, rtmp, stage_sh):
    c = lax.axis_index("c")
    s = lax.axis_index("s")
    w = s * NC + c
    seg = NP // NS

    zero16 = jnp.zeros((16,), jnp.float32)

    def zel(i, carry):
        hist[pl.ds(pl.multiple_of(i * 16, 16), 16)] = zero16
        return carry

    lax.fori_loop(0, NP // 16, zel, 0)

    base = w * EW
    ones16 = jnp.ones((16,), jnp.float32)

    def outer(k, carry):
        off = pl.multiple_of(base + k * IDXC, 8)
        pltpu.sync_copy(dst_hbm.at[pl.ds(off, IDXC)], idxb)

        def inner(i, carry2):
            v = idxb[pl.ds(pl.multiple_of(i * 16, 16), 16)]
            plsc.addupdate_scatter(hist, [v], ones16)
            return carry2

        lax.fori_loop(0, IDXC // 16, inner, 0)
        return carry

    lax.fori_loop(0, EW // IDXC, outer, 0)

    pltpu.sync_copy(hist, stage_sh.at[s])
    plsc.subcore_barrier()

    cbase = pl.multiple_of(s * seg, 8)
    pltpu.sync_copy(stage_sh.at[0, pl.ds(cbase, seg)], racc)
    for k in range(1, NS):
        pltpu.sync_copy(stage_sh.at[k, pl.ds(cbase, seg)], rtmp)

        def addel(i, carry):
            sl = pl.ds(pl.multiple_of(i * 16, 16), 16)
            racc[sl] = racc[sl] + rtmp[sl]
            return carry

        lax.fori_loop(0, seg // 16, addel, 0)
    pltpu.sync_copy(racc, out_hbm.at[c, pl.ds(cbase, seg)])


HD = D // 2


@functools.partial(
    pl.kernel,
    out_type=jax.ShapeDtypeStruct((NC, 2, NP, HD), jnp.float32),
    mesh=_mesh,
    scratch_types=[
        pltpu.VMEM((K, HD), jnp.float32),
        pltpu.VMEM((CH, K), jnp.int32),
        pltpu.VMEM((CH, K), jnp.int32),
        pltpu.VMEM_SHARED((NP, HD), jnp.float32),
        pltpu.VMEM_SHARED((NP, HD), jnp.float32),
        pltpu.SemaphoreType.DMA,
    ],
    compiler_params=pltpu.CompilerParams(
        needs_layout_passes=False, use_tc_tiling_on_sc=False
    ),
)
def _sc_seg(x_hbm, src_hbm, dst_hbm, out_hbm, rows, srcb, dstb,
            x_sh, acc_sh, sem):
    c = lax.axis_index("c")
    s = lax.axis_index("s")
    w = s * NC + c
    srows = NP // NS

    pltpu.sync_copy(src_hbm.at[w], srcb)
    pltpu.sync_copy(dst_hbm.at[w], dstb)

    for h in range(2):
        rbase = pl.multiple_of(s * srows, 8)
        pltpu.sync_copy(
            x_hbm.at[h, pl.ds(rbase, srows)],
            x_sh.at[pl.ds(rbase, srows)],
        )
        _zero_vmem_rows(rows, K, HD)
        for k in range(srows // K):
            off = pl.multiple_of(s * srows + k * K, 8)
            pltpu.sync_copy(rows, acc_sh.at[pl.ds(off, K)])
        plsc.subcore_barrier()

        def chunk(j, carry):
            pltpu.async_copy(x_sh.at[srcb.at[j]], rows, sem).wait()
            pltpu.sync_copy(rows, acc_sh.at[dstb.at[j]], add=True)
            return carry

        lax.fori_loop(0, CH, chunk, 0)
        plsc.subcore_barrier()

        for k in range(srows // K):
            off = pl.multiple_of(s * srows + k * K, 8)
            pltpu.sync_copy(acc_sh.at[pl.ds(off, K)], out_hbm.at[c, h, pl.ds(off, K)])
        plsc.subcore_barrier()


def _tc_in_body(x_ref, wc_ref, b_ref, n_ref, h0h_ref, xh_ref):
    x = x_ref[...]
    nc = n_ref[...]
    for hh in range(2):
        h = jnp.dot(x, wc_ref[hh], preferred_element_type=jnp.float32)
        h = h + b_ref[hh]
        h0h_ref[hh] = h
        xh_ref[hh] = h * nc


def _tc_in(x, wc, b2, norm1):
    return pl.pallas_call(
        _tc_in_body,
        grid=(GRID,),
        in_specs=[
            pl.BlockSpec((MB, D), lambda i: (i, 0)),
            pl.BlockSpec((2, D, HD), lambda i: (0, 0, 0)),
            pl.BlockSpec((2, HD), lambda i: (0, 0)),
            pl.BlockSpec((MB, 1), lambda i: (i, 0)),
        ],
        out_specs=[pl.BlockSpec((2, MB, HD), lambda i: (0, i, 0))] * 2,
        out_shape=[jax.ShapeDtypeStruct((2, NP, HD), jnp.float32)] * 2,
    )(x, wc, b2, norm1)


def _tc_mid_body(p_ref, n_ref, wq_ref, b_ref, h0h_ref, x1h_ref):
    nc = n_ref[...]
    y0 = (p_ref[0, 0] + p_ref[1, 0]) * nc
    y1 = (p_ref[0, 1] + p_ref[1, 1]) * nc
    for hh in range(2):
        t = (
            jnp.dot(y0, wq_ref[0, hh], preferred_element_type=jnp.float32)
            + jnp.dot(y1, wq_ref[1, hh], preferred_element_type=jnp.float32)
            + b_ref[hh]
        )
        t = jnp.maximum(t, 0.0) + h0h_ref[hh]
        x1h_ref[hh] = t * nc


def _tc_mid(p, norm1, wq, b2, h0h):
    return pl.pallas_call(
        _tc_mid_body,
        grid=(GRID,),
        in_specs=[
            pl.BlockSpec((NC, 2, MB, HD), lambda i: (0, 0, i, 0)),
            pl.BlockSpec((MB, 1), lambda i: (i, 0)),
            pl.BlockSpec((2, 2, HD, HD), lambda i: (0, 0, 0, 0)),
            pl.BlockSpec((2, HD), lambda i: (0, 0)),
            pl.BlockSpec((2, MB, HD), lambda i: (0, i, 0)),
        ],
        out_specs=pl.BlockSpec((2, MB, HD), lambda i: (0, i, 0)),
        out_shape=jax.ShapeDtypeStruct((2, NP, HD), jnp.float32),
    )(p, norm1, wq, b2, h0h)


def _tc_out_body(p_ref, n_ref, w1_ref, b1_ref, wo_ref, bo_ref, oh_ref):
    nc = n_ref[...]
    y0 = (p_ref[0, 0] + p_ref[1, 0]) * nc
    y1 = (p_ref[0, 1] + p_ref[1, 1]) * nc
    h2 = []
    for hh in range(2):
        t = (
            jnp.dot(y0, w1_ref[0, hh], preferred_element_type=jnp.float32)
            + jnp.dot(y1, w1_ref[1, hh], preferred_element_type=jnp.float32)
            + b1_ref[hh]
        )
        h2.append(jnp.maximum(t, 0.0))
    for hh in range(2):
        oh_ref[hh] = (
            jnp.dot(h2[0], wo_ref[0, hh], preferred_element_type=jnp.float32)
            + jnp.dot(h2[1], wo_ref[1, hh], preferred_element_type=jnp.float32)
            + bo_ref[hh]
        )


def _tc_out(p, norm1, w1q, b12, woq, bo2):
    return pl.pallas_call(
        _tc_out_body,
        grid=(GRID,),
        in_specs=[
            pl.BlockSpec((NC, 2, MB, HD), lambda i: (0, 0, i, 0)),
            pl.BlockSpec((MB, 1), lambda i: (i, 0)),
            pl.BlockSpec((2, 2, HD, HD), lambda i: (0, 0, 0, 0)),
            pl.BlockSpec((2, HD), lambda i: (0, 0)),
            pl.BlockSpec((2, 2, HD, HD), lambda i: (0, 0, 0, 0)),
            pl.BlockSpec((2, HD), lambda i: (0, 0)),
        ],
        out_specs=pl.BlockSpec((2, MB, HD), lambda i: (0, i, 0)),
        out_shape=jax.ShapeDtypeStruct((2, N, HD), jnp.float32),
    )(p, norm1, w1q, b12, woq, bo2)


def kernel(features, edge_index, W_in, b_in, W0, b0, W1, b1, W_out, b_out):
    src = edge_index[0].astype(jnp.int32)
    dst = edge_index[1].astype(jnp.int32)
    pad = EP - E
    srcp = jnp.concatenate([src, jnp.zeros((pad,), jnp.int32)])
    dstp = jnp.concatenate([dst, jnp.full((pad,), NP - 8, jnp.int32)])
    src3 = srcp.reshape(NW, CH, K)
    dst3 = dstp.reshape(NW, CH, K)

    degp = _sc_deg(dstp)
    deg = degp[0] + degp[1]
    norm1 = lax.rsqrt(jnp.maximum(deg, 1.0)).reshape(NP, 1)

    winc = W_in.reshape(D, 2, HD).transpose(1, 0, 2)
    w0q = W0.reshape(2, HD, 2, HD).transpose(0, 2, 1, 3)
    w1q = W1.reshape(2, HD, 2, HD).transpose(0, 2, 1, 3)
    woq = W_out.reshape(2, HD, 2, HD).transpose(0, 2, 1, 3)

    h0h, x0h = _tc_in(features, winc, b_in.reshape(2, HD), norm1)
    p = _sc_seg(x0h, src3, dst3)
    x1h = _tc_mid(p, norm1, w0q, b0.reshape(2, HD), h0h)
    p = _sc_seg(x1h, src3, dst3)
    oh = _tc_out(p, norm1, w1q, b1.reshape(2, HD), woq, b_out.reshape(2, HD))
    return jnp.concatenate([oh[0], oh[1]], axis=1)


# speedup vs baseline: 2.1943x; 1.3376x over previous
import functools

import jax
import jax.numpy as jnp
from jax import lax
from jax.experimental import pallas as pl
from jax.experimental.pallas import tpu as pltpu
from jax.experimental.pallas import tpu_sc as plsc

N = 10000
D = 128
E = 320000
NP = 10240
ROWS = NP // 128
NC = 2
NS = 16
NW = NC * NS
K = 64
CH = 160
EW = K * CH
EP = EW * NW
IDXC = 1280
MB = 400
GRID = N // MB

_mesh = plsc.VectorSubcoreMesh(
    core_axis_name="c", subcore_axis_name="s", num_cores=NC, num_subcores=NS
)


def _zero_vmem_rows(ref, nrows, ncols=128):
    zero16 = jnp.zeros((16,), jnp.float32)

    def zrow(r, carry):
        for j in range(ncols // 16):
            ref[r, pl.ds(j * 16, 16)] = zero16
        return carry

    lax.fori_loop(0, nrows, zrow, 0)


@functools.partial(
    pl.kernel,
    out_type=jax.ShapeDtypeStruct((NC, NP), jnp.float32),
    mesh=_mesh,
    scratch_types=[
        pltpu.VMEM((NP,), jnp.float32),
        pltpu.VMEM((IDXC,), jnp.int32),
        pltpu.VMEM((NP // NS,), jnp.float32),
        pltpu.VMEM((NP // NS,), jnp.float32),
        pltpu.VMEM_SHARED((NS, NP), jnp.float32),
    ],
    compiler_params=pltpu.CompilerParams(needs_layout_passes=False),
)
def _sc_deg(dst_hbm, out_hbm, hist, idxb, racc, rtmp, stage_sh):
    c = lax.axis_index("c")
    s = lax.axis_index("s")
    w = s * NC + c
    seg = NP // NS

    zero16 = jnp.zeros((16,), jnp.float32)

    def zel(i, carry):
        hist[pl.ds(pl.multiple_of(i * 16, 16), 16)] = zero16
        return carry

    lax.fori_loop(0, NP // 16, zel, 0)

    base = w * EW
    ones16 = jnp.ones((16,), jnp.float32)

    def outer(k, carry):
        off = pl.multiple_of(base + k * IDXC, 8)
        pltpu.sync_copy(dst_hbm.at[pl.ds(off, IDXC)], idxb)

        def inner(i, carry2):
            v = idxb[pl.ds(pl.multiple_of(i * 16, 16), 16)]
            plsc.addupdate_scatter(hist, [v], ones16)
            return carry2

        lax.fori_loop(0, IDXC // 16, inner, 0)
        return carry

    lax.fori_loop(0, EW // IDXC, outer, 0)

    pltpu.sync_copy(hist, stage_sh.at[s])
    plsc.subcore_barrier()

    cbase = pl.multiple_of(s * seg, 8)
    pltpu.sync_copy(stage_sh.at[0, pl.ds(cbase, seg)], racc)
    for k in range(1, NS):
        pltpu.sync_copy(stage_sh.at[k, pl.ds(cbase, seg)], rtmp)

        def addel(i, carry):
            sl = pl.ds(pl.multiple_of(i * 16, 16), 16)
            racc[sl] = racc[sl] + rtmp[sl]
            return carry

        lax.fori_loop(0, seg // 16, addel, 0)
    pltpu.sync_copy(racc, out_hbm.at[c, pl.ds(cbase, seg)])


HD = D // 2


@functools.partial(
    pl.kernel,
    out_type=jax.ShapeDtypeStruct((NC, 2, NP, HD), jnp.float32),
    mesh=_mesh,
    scratch_types=[
        pltpu.VMEM((K, HD), jnp.float32),
        pltpu.VMEM((K, HD), jnp.float32),
        pltpu.VMEM((K, HD), jnp.float32),
        pltpu.VMEM((K, HD), jnp.float32),
        pltpu.VMEM((CH, K), jnp.int32),
        pltpu.VMEM((CH, K), jnp.int32),
        pltpu.VMEM_SHARED((NP, HD), jnp.float32),
        pltpu.VMEM_SHARED((NP, HD), jnp.float32),
        pltpu.SemaphoreType.DMA,
        pltpu.SemaphoreType.DMA,
        pltpu.SemaphoreType.DMA,
        pltpu.SemaphoreType.DMA,
        pltpu.SemaphoreType.DMA,
        pltpu.SemaphoreType.DMA,
        pltpu.SemaphoreType.DMA,
        pltpu.SemaphoreType.DMA,
    ],
    compiler_params=pltpu.CompilerParams(
        needs_layout_passes=False, use_tc_tiling_on_sc=False
    ),
)
def _sc_seg(x_hbm, src_hbm, dst_hbm, out_hbm, b0, b1, b2, b3, srcb, dstb,
            x_sh, acc_sh, g0, g1, g2, g3, s0, s1, s2, s3):
    c = lax.axis_index("c")
    s = lax.axis_index("s")
    w = s * NC + c
    srows = NP // NS
    bufs = (b0, b1, b2, b3)
    gsem = (g0, g1, g2, g3)
    ssem = (s0, s1, s2, s3)

    pltpu.sync_copy(src_hbm.at[w], srcb)
    pltpu.sync_copy(dst_hbm.at[w], dstb)

    def gat(j, b):
        pltpu.async_copy(x_sh.at[srcb.at[j]], bufs[b], gsem[b])

    def gatw(j, b):
        pltpu.make_async_copy(x_sh.at[srcb.at[j]], bufs[b], gsem[b]).wait()

    def sca(j, b):
        pltpu.async_copy(bufs[b], acc_sh.at[dstb.at[j]], ssem[b], add=True)

    def scaw(j, b):
        pltpu.make_async_copy(bufs[b], acc_sh.at[dstb.at[j]], ssem[b]).wait()

    for h in range(2):
        rbase = pl.multiple_of(s * srows, 8)
        pltpu.sync_copy(
            x_hbm.at[h, pl.ds(rbase, srows)],
            x_sh.at[pl.ds(rbase, srows)],
        )
        _zero_vmem_rows(b0, K, HD)
        for k in range(srows // K):
            off = pl.multiple_of(s * srows + k * K, 8)
            pltpu.sync_copy(b0, acc_sh.at[pl.ds(off, K)])
        plsc.subcore_barrier()

        gat(0, 0)
        gat(1, 1)
        gatw(0, 0)
        sca(0, 0)
        gat(2, 2)
        gatw(1, 1)
        sca(1, 1)
        gat(3, 3)

        def step(i, carry):
            j0 = pl.multiple_of(i * 4 + 2, 2)
            for t in range(4):
                b = (2 + t) % 4
                j = j0 + t
                gatw(j, b)
                sca(j, b)
                scaw(j - 2, (b + 2) % 4)
                gat(j + 2, (b + 2) % 4)
            return carry

        lax.fori_loop(0, (CH - 4) // 4, step, 0)
        gatw(CH - 2, 2)
        sca(CH - 2, 2)
        gatw(CH - 1, 3)
        sca(CH - 1, 3)
        scaw(CH - 4, 0)
        scaw(CH - 3, 1)
        scaw(CH - 2, 2)
        scaw(CH - 1, 3)
        plsc.subcore_barrier()

        for k in range(srows // K):
            off = pl.multiple_of(s * srows + k * K, 8)
            pltpu.sync_copy(acc_sh.at[pl.ds(off, K)], out_hbm.at[c, h, pl.ds(off, K)])
        plsc.subcore_barrier()


def _tc_in_body(x_ref, wc_ref, b_ref, n_ref, h0h_ref, xh_ref):
    x = x_ref[...]
    nc = n_ref[...]
    for hh in range(2):
        h = jnp.dot(x, wc_ref[hh], preferred_element_type=jnp.float32)
        h = h + b_ref[hh]
        h0h_ref[hh] = h
        xh_ref[hh] = h * nc


def _tc_in(x, wc, b2, norm1):
    return pl.pallas_call(
        _tc_in_body,
        grid=(GRID,),
        in_specs=[
            pl.BlockSpec((MB, D), lambda i: (i, 0)),
            pl.BlockSpec((2, D, HD), lambda i: (0, 0, 0)),
            pl.BlockSpec((2, HD), lambda i: (0, 0)),
            pl.BlockSpec((MB, 1), lambda i: (i, 0)),
        ],
        out_specs=[pl.BlockSpec((2, MB, HD), lambda i: (0, i, 0))] * 2,
        out_shape=[jax.ShapeDtypeStruct((2, NP, HD), jnp.float32)] * 2,
    )(x, wc, b2, norm1)


def _tc_mid_body(p_ref, n_ref, wq_ref, b_ref, h0h_ref, x1h_ref):
    nc = n_ref[...]
    y0 = (p_ref[0, 0] + p_ref[1, 0]) * nc
    y1 = (p_ref[0, 1] + p_ref[1, 1]) * nc
    for hh in range(2):
        t = (
            jnp.dot(y0, wq_ref[0, hh], preferred_element_type=jnp.float32)
            + jnp.dot(y1, wq_ref[1, hh], preferred_element_type=jnp.float32)
            + b_ref[hh]
        )
        t = jnp.maximum(t, 0.0) + h0h_ref[hh]
        x1h_ref[hh] = t * nc


def _tc_mid(p, norm1, wq, b2, h0h):
    return pl.pallas_call(
        _tc_mid_body,
        grid=(GRID,),
        in_specs=[
            pl.BlockSpec((NC, 2, MB, HD), lambda i: (0, 0, i, 0)),
            pl.BlockSpec((MB, 1), lambda i: (i, 0)),
            pl.BlockSpec((2, 2, HD, HD), lambda i: (0, 0, 0, 0)),
            pl.BlockSpec((2, HD), lambda i: (0, 0)),
            pl.BlockSpec((2, MB, HD), lambda i: (0, i, 0)),
        ],
        out_specs=pl.BlockSpec((2, MB, HD), lambda i: (0, i, 0)),
        out_shape=jax.ShapeDtypeStruct((2, NP, HD), jnp.float32),
    )(p, norm1, wq, b2, h0h)


def _tc_out_body(p_ref, n_ref, w1_ref, b1_ref, wo_ref, bo_ref, oh_ref):
    nc = n_ref[...]
    y0 = (p_ref[0, 0] + p_ref[1, 0]) * nc
    y1 = (p_ref[0, 1] + p_ref[1, 1]) * nc
    h2 = []
    for hh in range(2):
        t = (
            jnp.dot(y0, w1_ref[0, hh], preferred_element_type=jnp.float32)
            + jnp.dot(y1, w1_ref[1, hh], preferred_element_type=jnp.float32)
            + b1_ref[hh]
        )
        h2.append(jnp.maximum(t, 0.0))
    for hh in range(2):
        oh_ref[hh] = (
            jnp.dot(h2[0], wo_ref[0, hh], preferred_element_type=jnp.float32)
            + jnp.dot(h2[1], wo_ref[1, hh], preferred_element_type=jnp.float32)
            + bo_ref[hh]
        )


def _tc_out(p, norm1, w1q, b12, woq, bo2):
    return pl.pallas_call(
        _tc_out_body,
        grid=(GRID,),
        in_specs=[
            pl.BlockSpec((NC, 2, MB, HD), lambda i: (0, 0, i, 0)),
            pl.BlockSpec((MB, 1), lambda i: (i, 0)),
            pl.BlockSpec((2, 2, HD, HD), lambda i: (0, 0, 0, 0)),
            pl.BlockSpec((2, HD), lambda i: (0, 0)),
            pl.BlockSpec((2, 2, HD, HD), lambda i: (0, 0, 0, 0)),
            pl.BlockSpec((2, HD), lambda i: (0, 0)),
        ],
        out_specs=pl.BlockSpec((2, MB, HD), lambda i: (0, i, 0)),
        out_shape=jax.ShapeDtypeStruct((2, N, HD), jnp.float32),
    )(p, norm1, w1q, b12, woq, bo2)


def kernel(features, edge_index, W_in, b_in, W0, b0, W1, b1, W_out, b_out):
    src = edge_index[0].astype(jnp.int32)
    dst = edge_index[1].astype(jnp.int32)
    pad = EP - E
    srcp = jnp.concatenate([src, jnp.zeros((pad,), jnp.int32)])
    dstp = jnp.concatenate([dst, jnp.full((pad,), NP - 8, jnp.int32)])
    src3 = srcp.reshape(NW, CH, K)
    dst3 = dstp.reshape(NW, CH, K)

    degp = _sc_deg(dstp)
    deg = degp[0] + degp[1]
    norm1 = lax.rsqrt(jnp.maximum(deg, 1.0)).reshape(NP, 1)

    winc = W_in.reshape(D, 2, HD).transpose(1, 0, 2)
    w0q = W0.reshape(2, HD, 2, HD).transpose(0, 2, 1, 3)
    w1q = W1.reshape(2, HD, 2, HD).transpose(0, 2, 1, 3)
    woq = W_out.reshape(2, HD, 2, HD).transpose(0, 2, 1, 3)

    h0h, x0h = _tc_in(features, winc, b_in.reshape(2, HD), norm1)
    p = _sc_seg(x0h, src3, dst3)
    x1h = _tc_mid(p, norm1, w0q, b0.reshape(2, HD), h0h)
    p = _sc_seg(x1h, src3, dst3)
    oh = _tc_out(p, norm1, w1q, b1.reshape(2, HD), woq, b_out.reshape(2, HD))
    return jnp.concatenate([oh[0], oh[1]], axis=1)


# speedup vs baseline: 2.3544x; 1.0730x over previous
import functools

import jax
import jax.numpy as jnp
from jax import lax
from jax.experimental import pallas as pl
from jax.experimental.pallas import tpu as pltpu
from jax.experimental.pallas import tpu_sc as plsc

N = 10000
D = 128
E = 320000
NP = 10240
ROWS = NP // 128
NC = 2
NS = 16
NW = NC * NS
K = 64
CH = 160
EW = K * CH
EP = EW * NW
IDXC = 1280
MB = 2000
GRID = N // MB

_mesh = plsc.VectorSubcoreMesh(
    core_axis_name="c", subcore_axis_name="s", num_cores=NC, num_subcores=NS
)


def _zero_vmem_rows(ref, nrows, ncols=128):
    zero16 = jnp.zeros((16,), jnp.float32)

    def zrow(r, carry):
        for j in range(ncols // 16):
            ref[r, pl.ds(j * 16, 16)] = zero16
        return carry

    lax.fori_loop(0, nrows, zrow, 0)


@functools.partial(
    pl.kernel,
    out_type=jax.ShapeDtypeStruct((NC, NP), jnp.float32),
    mesh=_mesh,
    scratch_types=[
        pltpu.VMEM((NP,), jnp.float32),
        pltpu.VMEM((IDXC,), jnp.int32),
        pltpu.VMEM((NP // NS,), jnp.float32),
        pltpu.VMEM((NP // NS,), jnp.float32),
        pltpu.VMEM_SHARED((NS, NP), jnp.float32),
    ],
    compiler_params=pltpu.CompilerParams(needs_layout_passes=False),
)
def _sc_deg(dst_hbm, out_hbm, hist, idxb, racc, rtmp, stage_sh):
    c = lax.axis_index("c")
    s = lax.axis_index("s")
    w = s * NC + c
    seg = NP // NS

    zero16 = jnp.zeros((16,), jnp.float32)

    def zel(i, carry):
        hist[pl.ds(pl.multiple_of(i * 16, 16), 16)] = zero16
        return carry

    lax.fori_loop(0, NP // 16, zel, 0)

    base = w * EW
    ones16 = jnp.ones((16,), jnp.float32)

    def outer(k, carry):
        off = pl.multiple_of(base + k * IDXC, 8)
        pltpu.sync_copy(dst_hbm.at[pl.ds(off, IDXC)], idxb)

        def inner(i, carry2):
            v = idxb[pl.ds(pl.multiple_of(i * 16, 16), 16)]
            plsc.addupdate_scatter(hist, [v], ones16)
            return carry2

        lax.fori_loop(0, IDXC // 16, inner, 0)
        return carry

    lax.fori_loop(0, EW // IDXC, outer, 0)

    pltpu.sync_copy(hist, stage_sh.at[s])
    plsc.subcore_barrier()

    cbase = pl.multiple_of(s * seg, 8)
    pltpu.sync_copy(stage_sh.at[0, pl.ds(cbase, seg)], racc)
    for k in range(1, NS):
        pltpu.sync_copy(stage_sh.at[k, pl.ds(cbase, seg)], rtmp)

        def addel(i, carry):
            sl = pl.ds(pl.multiple_of(i * 16, 16), 16)
            racc[sl] = racc[sl] + rtmp[sl]
            return carry

        lax.fori_loop(0, seg // 16, addel, 0)
    pltpu.sync_copy(racc, out_hbm.at[c, pl.ds(cbase, seg)])


HD = D // 2


@functools.partial(
    pl.kernel,
    out_type=jax.ShapeDtypeStruct((NC, 2, NP, HD), jnp.float32),
    mesh=_mesh,
    scratch_types=[
        pltpu.VMEM((K, HD), jnp.float32),
        pltpu.VMEM((K, HD), jnp.float32),
        pltpu.VMEM((K, HD), jnp.float32),
        pltpu.VMEM((K, HD), jnp.float32),
        pltpu.VMEM((CH, K), jnp.int32),
        pltpu.VMEM((CH, K), jnp.int32),
        pltpu.VMEM_SHARED((NP, HD), jnp.float32),
        pltpu.VMEM_SHARED((NP, HD), jnp.float32),
        pltpu.SemaphoreType.DMA,
        pltpu.SemaphoreType.DMA,
        pltpu.SemaphoreType.DMA,
        pltpu.SemaphoreType.DMA,
        pltpu.SemaphoreType.DMA,
        pltpu.SemaphoreType.DMA,
        pltpu.SemaphoreType.DMA,
        pltpu.SemaphoreType.DMA,
    ],
    compiler_params=pltpu.CompilerParams(
        needs_layout_passes=False, use_tc_tiling_on_sc=False
    ),
)
def _sc_seg(x_hbm, src_hbm, dst_hbm, out_hbm, b0, b1, b2, b3, srcb, dstb,
            x_sh, acc_sh, g0, g1, g2, g3, s0, s1, s2, s3):
    c = lax.axis_index("c")
    s = lax.axis_index("s")
    w = s * NC + c
    srows = NP // NS
    bufs = (b0, b1, b2, b3)
    gsem = (g0, g1, g2, g3)
    ssem = (s0, s1, s2, s3)

    pltpu.sync_copy(src_hbm.at[w], srcb)
    pltpu.sync_copy(dst_hbm.at[w], dstb)

    def gat(j, b):
        pltpu.async_copy(x_sh.at[srcb.at[j]], bufs[b], gsem[b])

    def gatw(j, b):
        pltpu.make_async_copy(x_sh.at[srcb.at[j]], bufs[b], gsem[b]).wait()

    def sca(j, b):
        pltpu.async_copy(bufs[b], acc_sh.at[dstb.at[j]], ssem[b], add=True)

    def scaw(j, b):
        pltpu.make_async_copy(bufs[b], acc_sh.at[dstb.at[j]], ssem[b]).wait()

    for h in range(2):
        rbase = pl.multiple_of(s * srows, 8)
        pltpu.sync_copy(
            x_hbm.at[h, pl.ds(rbase, srows)],
            x_sh.at[pl.ds(rbase, srows)],
        )
        _zero_vmem_rows(b0, K, HD)
        for k in range(srows // K):
            off = pl.multiple_of(s * srows + k * K, 8)
            pltpu.sync_copy(b0, acc_sh.at[pl.ds(off, K)])
        plsc.subcore_barrier()

        gat(0, 0)
        gat(1, 1)
        gatw(0, 0)
        sca(0, 0)
        gat(2, 2)
        gatw(1, 1)
        sca(1, 1)
        gat(3, 3)

        def step(i, carry):
            j0 = pl.multiple_of(i * 4 + 2, 2)
            for t in range(4):
                b = (2 + t) % 4
                j = j0 + t
                gatw(j, b)
                sca(j, b)
                scaw(j - 2, (b + 2) % 4)
                gat(j + 2, (b + 2) % 4)
            return carry

        lax.fori_loop(0, (CH - 4) // 4, step, 0)
        gatw(CH - 2, 2)
        sca(CH - 2, 2)
        gatw(CH - 1, 3)
        sca(CH - 1, 3)
        scaw(CH - 4, 0)
        scaw(CH - 3, 1)
        scaw(CH - 2, 2)
        scaw(CH - 1, 3)
        plsc.subcore_barrier()

        for k in range(srows // K):
            off = pl.multiple_of(s * srows + k * K, 8)
            pltpu.sync_copy(acc_sh.at[pl.ds(off, K)], out_hbm.at[c, h, pl.ds(off, K)])
        plsc.subcore_barrier()


def _tc_in_body(x_ref, wc_ref, b_ref, n_ref, h0h_ref, xh_ref):
    x = x_ref[...]
    nc = n_ref[...]
    for hh in range(2):
        h = jnp.dot(x, wc_ref[hh], preferred_element_type=jnp.float32)
        h = h + b_ref[hh]
        h0h_ref[hh] = h
        xh_ref[hh] = h * nc


def _tc_in(x, wc, b2, norm1):
    return pl.pallas_call(
        _tc_in_body,
        grid=(GRID,),
        in_specs=[
            pl.BlockSpec((MB, D), lambda i: (i, 0)),
            pl.BlockSpec((2, D, HD), lambda i: (0, 0, 0)),
            pl.BlockSpec((2, HD), lambda i: (0, 0)),
            pl.BlockSpec((MB, 1), lambda i: (i, 0)),
        ],
        out_specs=[pl.BlockSpec((2, MB, HD), lambda i: (0, i, 0))] * 2,
        out_shape=[jax.ShapeDtypeStruct((2, NP, HD), jnp.float32)] * 2,
    )(x, wc, b2, norm1)


def _tc_mid_body(p_ref, n_ref, wq_ref, b_ref, h0h_ref, x1h_ref):
    nc = n_ref[...]
    y0 = (p_ref[0, 0] + p_ref[1, 0]) * nc
    y1 = (p_ref[0, 1] + p_ref[1, 1]) * nc
    for hh in range(2):
        t = (
            jnp.dot(y0, wq_ref[0, hh], preferred_element_type=jnp.float32)
            + jnp.dot(y1, wq_ref[1, hh], preferred_element_type=jnp.float32)
            + b_ref[hh]
        )
        t = jnp.maximum(t, 0.0) + h0h_ref[hh]
        x1h_ref[hh] = t * nc


def _tc_mid(p, norm1, wq, b2, h0h):
    return pl.pallas_call(
        _tc_mid_body,
        grid=(GRID,),
        in_specs=[
            pl.BlockSpec((NC, 2, MB, HD), lambda i: (0, 0, i, 0)),
            pl.BlockSpec((MB, 1), lambda i: (i, 0)),
            pl.BlockSpec((2, 2, HD, HD), lambda i: (0, 0, 0, 0)),
            pl.BlockSpec((2, HD), lambda i: (0, 0)),
            pl.BlockSpec((2, MB, HD), lambda i: (0, i, 0)),
        ],
        out_specs=pl.BlockSpec((2, MB, HD), lambda i: (0, i, 0)),
        out_shape=jax.ShapeDtypeStruct((2, NP, HD), jnp.float32),
    )(p, norm1, wq, b2, h0h)


def _tc_out_body(p_ref, n_ref, w1_ref, b1_ref, wo_ref, bo_ref, oh_ref):
    nc = n_ref[...]
    y0 = (p_ref[0, 0] + p_ref[1, 0]) * nc
    y1 = (p_ref[0, 1] + p_ref[1, 1]) * nc
    h2 = []
    for hh in range(2):
        t = (
            jnp.dot(y0, w1_ref[0, hh], preferred_element_type=jnp.float32)
            + jnp.dot(y1, w1_ref[1, hh], preferred_element_type=jnp.float32)
            + b1_ref[hh]
        )
        h2.append(jnp.maximum(t, 0.0))
    for hh in range(2):
        oh_ref[hh] = (
            jnp.dot(h2[0], wo_ref[0, hh], preferred_element_type=jnp.float32)
            + jnp.dot(h2[1], wo_ref[1, hh], preferred_element_type=jnp.float32)
            + bo_ref[hh]
        )


def _tc_out(p, norm1, w1q, b12, woq, bo2):
    return pl.pallas_call(
        _tc_out_body,
        grid=(GRID,),
        in_specs=[
            pl.BlockSpec((NC, 2, MB, HD), lambda i: (0, 0, i, 0)),
            pl.BlockSpec((MB, 1), lambda i: (i, 0)),
            pl.BlockSpec((2, 2, HD, HD), lambda i: (0, 0, 0, 0)),
            pl.BlockSpec((2, HD), lambda i: (0, 0)),
            pl.BlockSpec((2, 2, HD, HD), lambda i: (0, 0, 0, 0)),
            pl.BlockSpec((2, HD), lambda i: (0, 0)),
        ],
        out_specs=pl.BlockSpec((2, MB, HD), lambda i: (0, i, 0)),
        out_shape=jax.ShapeDtypeStruct((2, N, HD), jnp.float32),
    )(p, norm1, w1q, b12, woq, bo2)


def kernel(features, edge_index, W_in, b_in, W0, b0, W1, b1, W_out, b_out):
    src = edge_index[0].astype(jnp.int32)
    dst = edge_index[1].astype(jnp.int32)
    pad = EP - E
    srcp = jnp.concatenate([src, jnp.zeros((pad,), jnp.int32)])
    dstp = jnp.concatenate([dst, jnp.full((pad,), NP - 8, jnp.int32)])
    src3 = srcp.reshape(NW, CH, K)
    dst3 = dstp.reshape(NW, CH, K)

    degp = _sc_deg(dstp)
    deg = degp[0] + degp[1]
    norm1 = lax.rsqrt(jnp.maximum(deg, 1.0)).reshape(NP, 1)

    winc = W_in.reshape(D, 2, HD).transpose(1, 0, 2)
    w0q = W0.reshape(2, HD, 2, HD).transpose(0, 2, 1, 3)
    w1q = W1.reshape(2, HD, 2, HD).transpose(0, 2, 1, 3)
    woq = W_out.reshape(2, HD, 2, HD).transpose(0, 2, 1, 3)

    h0h, x0h = _tc_in(features, winc, b_in.reshape(2, HD), norm1)
    p = _sc_seg(x0h, src3, dst3)
    x1h = _tc_mid(p, norm1, w0q, b0.reshape(2, HD), h0h)
    p = _sc_seg(x1h, src3, dst3)
    oh = _tc_out(p, norm1, w1q, b1.reshape(2, HD), woq, b_out.reshape(2, HD))
    return jnp.concatenate([oh[0], oh[1]], axis=1)


# speedup vs baseline: 2.3688x; 1.0061x over previous
import functools

import jax
import jax.numpy as jnp
from jax import lax
from jax.experimental import pallas as pl
from jax.experimental.pallas import tpu as pltpu
from jax.experimental.pallas import tpu_sc as plsc

N = 10000
D = 128
E = 320000
NP = 10240
ROWS = NP // 128
NC = 2
NS = 16
NW = NC * NS
K = 64
CH = 160
EW = K * CH
EP = EW * NW
IDXC = 1280
MB = 2000
GRID = N // MB

_mesh = plsc.VectorSubcoreMesh(
    core_axis_name="c", subcore_axis_name="s", num_cores=NC, num_subcores=NS
)


def _zero_vmem_rows(ref, nrows, ncols=128):
    zero16 = jnp.zeros((16,), jnp.float32)

    def zrow(r, carry):
        for j in range(ncols // 16):
            ref[r, pl.ds(j * 16, 16)] = zero16
        return carry

    lax.fori_loop(0, nrows, zrow, 0)


@functools.partial(
    pl.kernel,
    out_type=jax.ShapeDtypeStruct((NC, NP), jnp.float32),
    mesh=_mesh,
    scratch_types=[
        pltpu.VMEM((NP,), jnp.float32),
        pltpu.VMEM((IDXC,), jnp.int32),
        pltpu.VMEM((NP // NS,), jnp.float32),
        pltpu.VMEM((NP // NS,), jnp.float32),
        pltpu.VMEM_SHARED((NS, NP), jnp.float32),
    ],
    compiler_params=pltpu.CompilerParams(needs_layout_passes=False),
)
def _sc_deg(dst_hbm, out_hbm, hist, idxb, racc, rtmp, stage_sh):
    c = lax.axis_index("c")
    s = lax.axis_index("s")
    w = s * NC + c
    seg = NP // NS

    zero16 = jnp.zeros((16,), jnp.float32)

    def zel(i, carry):
        hist[pl.ds(pl.multiple_of(i * 16, 16), 16)] = zero16
        return carry

    lax.fori_loop(0, NP // 16, zel, 0)

    base = w * EW
    ones16 = jnp.ones((16,), jnp.float32)

    def outer(k, carry):
        off = pl.multiple_of(base + k * IDXC, 8)
        pltpu.sync_copy(dst_hbm.at[pl.ds(off, IDXC)], idxb)

        def inner(i, carry2):
            v = idxb[pl.ds(pl.multiple_of(i * 16, 16), 16)]
            plsc.addupdate_scatter(hist, [v], ones16)
            return carry2

        lax.fori_loop(0, IDXC // 16, inner, 0)
        return carry

    lax.fori_loop(0, EW // IDXC, outer, 0)

    pltpu.sync_copy(hist, stage_sh.at[s])
    plsc.subcore_barrier()

    cbase = pl.multiple_of(s * seg, 8)
    pltpu.sync_copy(stage_sh.at[0, pl.ds(cbase, seg)], racc)
    for k in range(1, NS):
        pltpu.sync_copy(stage_sh.at[k, pl.ds(cbase, seg)], rtmp)

        def addel(i, carry):
            sl = pl.ds(pl.multiple_of(i * 16, 16), 16)
            racc[sl] = racc[sl] + rtmp[sl]
            return carry

        lax.fori_loop(0, seg // 16, addel, 0)
    pltpu.sync_copy(racc, out_hbm.at[c, pl.ds(cbase, seg)])


HD = D // 2


@functools.partial(
    pl.kernel,
    out_type=jax.ShapeDtypeStruct((NC, 2, NP, HD), jnp.float32),
    mesh=_mesh,
    scratch_types=[
        pltpu.VMEM((K, HD), jnp.float32),
        pltpu.VMEM((K, HD), jnp.float32),
        pltpu.VMEM((K, HD), jnp.float32),
        pltpu.VMEM((K, HD), jnp.float32),
        pltpu.VMEM((K, HD), jnp.float32),
        pltpu.VMEM((K, HD), jnp.float32),
        pltpu.VMEM((CH, K), jnp.int32),
        pltpu.VMEM((CH, K), jnp.int32),
        pltpu.VMEM_SHARED((NP, HD), jnp.float32),
        pltpu.VMEM_SHARED((NP, HD), jnp.float32),
        pltpu.SemaphoreType.DMA,
        pltpu.SemaphoreType.DMA,
        pltpu.SemaphoreType.DMA,
        pltpu.SemaphoreType.DMA,
        pltpu.SemaphoreType.DMA,
        pltpu.SemaphoreType.DMA,
        pltpu.SemaphoreType.DMA,
        pltpu.SemaphoreType.DMA,
        pltpu.SemaphoreType.DMA,
        pltpu.SemaphoreType.DMA,
        pltpu.SemaphoreType.DMA,
        pltpu.SemaphoreType.DMA,
    ],
    compiler_params=pltpu.CompilerParams(
        needs_layout_passes=False, use_tc_tiling_on_sc=False
    ),
)
def _sc_seg(x_hbm, src_hbm, dst_hbm, out_hbm, b0, b1, b2, b3, b4, b5,
            srcb, dstb, x_sh, acc_sh,
            g0, g1, g2, g3, g4, g5, s0, s1, s2, s3, s4, s5):
    c = lax.axis_index("c")
    s = lax.axis_index("s")
    w = s * NC + c
    srows = NP // NS
    bufs = (b0, b1, b2, b3, b4, b5)
    gsem = (g0, g1, g2, g3, g4, g5)
    ssem = (s0, s1, s2, s3, s4, s5)

    pltpu.sync_copy(src_hbm.at[w], srcb)
    pltpu.sync_copy(dst_hbm.at[w], dstb)

    def gat(j, b):
        pltpu.async_copy(x_sh.at[srcb.at[j]], bufs[b], gsem[b])

    def gatw(j, b):
        pltpu.make_async_copy(x_sh.at[srcb.at[j]], bufs[b], gsem[b]).wait()

    def sca(j, b):
        pltpu.async_copy(bufs[b], acc_sh.at[dstb.at[j]], ssem[b], add=True)

    def scaw(j, b):
        pltpu.make_async_copy(bufs[b], acc_sh.at[dstb.at[j]], ssem[b]).wait()

    for h in range(2):
        rbase = pl.multiple_of(s * srows, 8)
        pltpu.sync_copy(
            x_hbm.at[h, pl.ds(rbase, srows)],
            x_sh.at[pl.ds(rbase, srows)],
        )
        _zero_vmem_rows(b0, K, HD)
        for k in range(srows // K):
            off = pl.multiple_of(s * srows + k * K, 8)
            pltpu.sync_copy(b0, acc_sh.at[pl.ds(off, K)])
        plsc.subcore_barrier()

        for q in range(4):
            gat(q, q)
        gatw(0, 0)
        sca(0, 0)
        gat(4, 4)
        gatw(1, 1)
        sca(1, 1)
        gat(5, 5)

        def step(i, carry):
            j0 = pl.multiple_of(i * 6 + 2, 2)
            for t in range(6):
                b = (2 + t) % 6
                j = j0 + t
                gatw(j, b)
                sca(j, b)
                scaw(j - 2, (b + 4) % 6)
                gat(jnp.minimum(j + 4, CH - 1), (b + 4) % 6)
            return carry

        lax.fori_loop(0, (CH - 4) // 6, step, 0)
        gatw(CH - 2, (CH - 2) % 6)
        sca(CH - 2, (CH - 2) % 6)
        gatw(CH - 1, (CH - 1) % 6)
        sca(CH - 1, (CH - 1) % 6)
        gatw(CH - 1, CH % 6)
        gatw(CH - 1, (CH + 1) % 6)
        scaw(CH - 4, (CH - 4) % 6)
        scaw(CH - 3, (CH - 3) % 6)
        scaw(CH - 2, (CH - 2) % 6)
        scaw(CH - 1, (CH - 1) % 6)
        plsc.subcore_barrier()

        for k in range(srows // K):
            off = pl.multiple_of(s * srows + k * K, 8)
            pltpu.sync_copy(acc_sh.at[pl.ds(off, K)], out_hbm.at[c, h, pl.ds(off, K)])
        plsc.subcore_barrier()


def _tc_in_body(x_ref, wc_ref, b_ref, n_ref, h0h_ref, xh_ref):
    x = x_ref[...]
    nc = n_ref[...]
    for hh in range(2):
        h = jnp.dot(x, wc_ref[hh], preferred_element_type=jnp.float32)
        h = h + b_ref[hh]
        h0h_ref[hh] = h
        xh_ref[hh] = h * nc


def _tc_in(x, wc, b2, norm1):
    return pl.pallas_call(
        _tc_in_body,
        grid=(GRID,),
        in_specs=[
            pl.BlockSpec((MB, D), lambda i: (i, 0)),
            pl.BlockSpec((2, D, HD), lambda i: (0, 0, 0)),
            pl.BlockSpec((2, HD), lambda i: (0, 0)),
            pl.BlockSpec((MB, 1), lambda i: (i, 0)),
        ],
        out_specs=[pl.BlockSpec((2, MB, HD), lambda i: (0, i, 0))] * 2,
        out_shape=[jax.ShapeDtypeStruct((2, NP, HD), jnp.float32)] * 2,
    )(x, wc, b2, norm1)


def _tc_mid_body(p_ref, n_ref, wq_ref, b_ref, h0h_ref, x1h_ref):
    nc = n_ref[...]
    y0 = (p_ref[0, 0] + p_ref[1, 0]) * nc
    y1 = (p_ref[0, 1] + p_ref[1, 1]) * nc
    for hh in range(2):
        t = (
            jnp.dot(y0, wq_ref[0, hh], preferred_element_type=jnp.float32)
            + jnp.dot(y1, wq_ref[1, hh], preferred_element_type=jnp.float32)
            + b_ref[hh]
        )
        t = jnp.maximum(t, 0.0) + h0h_ref[hh]
        x1h_ref[hh] = t * nc


def _tc_mid(p, norm1, wq, b2, h0h):
    return pl.pallas_call(
        _tc_mid_body,
        grid=(GRID,),
        in_specs=[
            pl.BlockSpec((NC, 2, MB, HD), lambda i: (0, 0, i, 0)),
            pl.BlockSpec((MB, 1), lambda i: (i, 0)),
            pl.BlockSpec((2, 2, HD, HD), lambda i: (0, 0, 0, 0)),
            pl.BlockSpec((2, HD), lambda i: (0, 0)),
            pl.BlockSpec((2, MB, HD), lambda i: (0, i, 0)),
        ],
        out_specs=pl.BlockSpec((2, MB, HD), lambda i: (0, i, 0)),
        out_shape=jax.ShapeDtypeStruct((2, NP, HD), jnp.float32),
    )(p, norm1, wq, b2, h0h)


def _tc_out_body(p_ref, n_ref, w1_ref, b1_ref, wo_ref, bo_ref, oh_ref):
    nc = n_ref[...]
    y0 = (p_ref[0, 0] + p_ref[1, 0]) * nc
    y1 = (p_ref[0, 1] + p_ref[1, 1]) * nc
    h2 = []
    for hh in range(2):
        t = (
            jnp.dot(y0, w1_ref[0, hh], preferred_element_type=jnp.float32)
            + jnp.dot(y1, w1_ref[1, hh], preferred_element_type=jnp.float32)
            + b1_ref[hh]
        )
        h2.append(jnp.maximum(t, 0.0))
    for hh in range(2):
        oh_ref[hh] = (
            jnp.dot(h2[0], wo_ref[0, hh], preferred_element_type=jnp.float32)
            + jnp.dot(h2[1], wo_ref[1, hh], preferred_element_type=jnp.float32)
            + bo_ref[hh]
        )


def _tc_out(p, norm1, w1q, b12, woq, bo2):
    return pl.pallas_call(
        _tc_out_body,
        grid=(GRID,),
        in_specs=[
            pl.BlockSpec((NC, 2, MB, HD), lambda i: (0, 0, i, 0)),
            pl.BlockSpec((MB, 1), lambda i: (i, 0)),
            pl.BlockSpec((2, 2, HD, HD), lambda i: (0, 0, 0, 0)),
            pl.BlockSpec((2, HD), lambda i: (0, 0)),
            pl.BlockSpec((2, 2, HD, HD), lambda i: (0, 0, 0, 0)),
            pl.BlockSpec((2, HD), lambda i: (0, 0)),
        ],
        out_specs=pl.BlockSpec((2, MB, HD), lambda i: (0, i, 0)),
        out_shape=jax.ShapeDtypeStruct((2, N, HD), jnp.float32),
    )(p, norm1, w1q, b12, woq, bo2)


def kernel(features, edge_index, W_in, b_in, W0, b0, W1, b1, W_out, b_out):
    src = edge_index[0].astype(jnp.int32)
    dst = edge_index[1].astype(jnp.int32)
    pad = EP - E
    srcp = jnp.concatenate([src, jnp.zeros((pad,), jnp.int32)])
    dstp = jnp.concatenate([dst, jnp.full((pad,), NP - 8, jnp.int32)])
    src3 = srcp.reshape(NW, CH, K)
    dst3 = dstp.reshape(NW, CH, K)

    degp = _sc_deg(dstp)
    deg = degp[0] + degp[1]
    norm1 = lax.rsqrt(jnp.maximum(deg, 1.0)).reshape(NP, 1)

    winc = W_in.reshape(D, 2, HD).transpose(1, 0, 2)
    w0q = W0.reshape(2, HD, 2, HD).transpose(0, 2, 1, 3)
    w1q = W1.reshape(2, HD, 2, HD).transpose(0, 2, 1, 3)
    woq = W_out.reshape(2, HD, 2, HD).transpose(0, 2, 1, 3)

    h0h, x0h = _tc_in(features, winc, b_in.reshape(2, HD), norm1)
    p = _sc_seg(x0h, src3, dst3)
    x1h = _tc_mid(p, norm1, w0q, b0.reshape(2, HD), h0h)
    p = _sc_seg(x1h, src3, dst3)
    oh = _tc_out(p, norm1, w1q, b1.reshape(2, HD), woq, b_out.reshape(2, HD))
    return jnp.concatenate([oh[0], oh[1]], axis=1)


# speedup vs baseline: 2.6611x; 1.1234x over previous
import functools

import jax
import jax.numpy as jnp
from jax import lax
from jax.experimental import pallas as pl
from jax.experimental.pallas import tpu as pltpu
from jax.experimental.pallas import tpu_sc as plsc

N = 10000
D = 128
E = 320000
NP = 10240
ROWS = NP // 128
NC = 2
NS = 16
NW = NC * NS
K = 64
CH = 160
EW = K * CH
EP = EW * NW
IDXC = 1280
MB = 2000
GRID = N // MB

_mesh = plsc.VectorSubcoreMesh(
    core_axis_name="c", subcore_axis_name="s", num_cores=NC, num_subcores=NS
)


def _zero_vmem_rows(ref, nrows, ncols=128):
    zero16 = jnp.zeros((16,), jnp.float32)

    def zrow(r, carry):
        for j in range(ncols // 16):
            ref[r, pl.ds(j * 16, 16)] = zero16
        return carry

    lax.fori_loop(0, nrows, zrow, 0)


@functools.partial(
    pl.kernel,
    out_type=jax.ShapeDtypeStruct((NC, NP), jnp.float32),
    mesh=_mesh,
    scratch_types=[
        pltpu.VMEM((NP,), jnp.float32),
        pltpu.VMEM((IDXC,), jnp.int32),
        pltpu.VMEM((NP // NS,), jnp.float32),
        pltpu.VMEM((NP // NS,), jnp.float32),
        pltpu.VMEM_SHARED((NS, NP), jnp.float32),
    ],
    compiler_params=pltpu.CompilerParams(needs_layout_passes=False),
)
def _sc_deg(dst_hbm, out_hbm, hist, idxb, racc, rtmp, stage_sh):
    c = lax.axis_index("c")
    s = lax.axis_index("s")
    w = s * NC + c
    seg = NP // NS

    zero16 = jnp.zeros((16,), jnp.float32)

    def zel(i, carry):
        hist[pl.ds(pl.multiple_of(i * 16, 16), 16)] = zero16
        return carry

    lax.fori_loop(0, NP // 16, zel, 0)

    base = w * EW
    ones16 = jnp.ones((16,), jnp.float32)

    def outer(k, carry):
        off = pl.multiple_of(base + k * IDXC, 8)
        pltpu.sync_copy(dst_hbm.at[pl.ds(off, IDXC)], idxb)

        def inner(i, carry2):
            v = idxb[pl.ds(pl.multiple_of(i * 16, 16), 16)]
            plsc.addupdate_scatter(hist, [v], ones16)
            return carry2

        lax.fori_loop(0, IDXC // 16, inner, 0)
        return carry

    lax.fori_loop(0, EW // IDXC, outer, 0)

    pltpu.sync_copy(hist, stage_sh.at[s])
    plsc.subcore_barrier()

    cbase = pl.multiple_of(s * seg, 8)
    pltpu.sync_copy(stage_sh.at[0, pl.ds(cbase, seg)], racc)
    for k in range(1, NS):
        pltpu.sync_copy(stage_sh.at[k, pl.ds(cbase, seg)], rtmp)

        def addel(i, carry):
            sl = pl.ds(pl.multiple_of(i * 16, 16), 16)
            racc[sl] = racc[sl] + rtmp[sl]
            return carry

        lax.fori_loop(0, seg // 16, addel, 0)
    pltpu.sync_copy(racc, out_hbm.at[c, pl.ds(cbase, seg)])


HD = D // 2


@functools.partial(
    pl.kernel,
    out_type=jax.ShapeDtypeStruct((2, NP, HD), jnp.float32),
    mesh=_mesh,
    scratch_types=[
        pltpu.VMEM((K, HD), jnp.float32),
        pltpu.VMEM((K, HD), jnp.float32),
        pltpu.VMEM((K, HD), jnp.float32),
        pltpu.VMEM((K, HD), jnp.float32),
        pltpu.VMEM((K, HD), jnp.float32),
        pltpu.VMEM((K, HD), jnp.float32),
        pltpu.VMEM((CH, K), jnp.int32),
        pltpu.VMEM((CH, K), jnp.int32),
        pltpu.VMEM_SHARED((NP, HD), jnp.float32),
        pltpu.VMEM_SHARED((NP, HD), jnp.float32),
        pltpu.SemaphoreType.DMA,
        pltpu.SemaphoreType.DMA,
        pltpu.SemaphoreType.DMA,
        pltpu.SemaphoreType.DMA,
        pltpu.SemaphoreType.DMA,
        pltpu.SemaphoreType.DMA,
        pltpu.SemaphoreType.DMA,
        pltpu.SemaphoreType.DMA,
        pltpu.SemaphoreType.DMA,
        pltpu.SemaphoreType.DMA,
        pltpu.SemaphoreType.DMA,
        pltpu.SemaphoreType.DMA,
    ],
    compiler_params=pltpu.CompilerParams(
        needs_layout_passes=False, use_tc_tiling_on_sc=False
    ),
)
def _sc_seg(x_hbm, src_hbm, dst_hbm, out_hbm, b0, b1, b2, b3, b4, b5,
            srcb, dstb, x_sh, acc_sh,
            g0, g1, g2, g3, g4, g5, s0, s1, s2, s3, s4, s5):
    c = lax.axis_index("c")
    s = lax.axis_index("s")
    srows = NP // NS
    bufs = (b0, b1, b2, b3, b4, b5)
    gsem = (g0, g1, g2, g3, g4, g5)
    ssem = (s0, s1, s2, s3, s4, s5)

    def gat(j, b):
        pltpu.async_copy(x_sh.at[srcb.at[j]], bufs[b], gsem[b])

    def gatw(j, b):
        pltpu.make_async_copy(x_sh.at[srcb.at[j]], bufs[b], gsem[b]).wait()

    def sca(j, b):
        pltpu.async_copy(bufs[b], acc_sh.at[dstb.at[j]], ssem[b], add=True)

    def scaw(j, b):
        pltpu.make_async_copy(bufs[b], acc_sh.at[dstb.at[j]], ssem[b]).wait()

    rbase = pl.multiple_of(s * srows, 8)
    pltpu.sync_copy(
        x_hbm.at[c, pl.ds(rbase, srows)],
        x_sh.at[pl.ds(rbase, srows)],
    )
    _zero_vmem_rows(b0, K, HD)
    for k in range(srows // K):
        off = pl.multiple_of(s * srows + k * K, 8)
        pltpu.sync_copy(b0, acc_sh.at[pl.ds(off, K)])
    plsc.subcore_barrier()

    for phase in range(2):
        pltpu.sync_copy(src_hbm.at[s, phase], srcb)
        pltpu.sync_copy(dst_hbm.at[s, phase], dstb)

        for q in range(4):
            gat(q, q)
        gatw(0, 0)
        sca(0, 0)
        gat(4, 4)
        gatw(1, 1)
        sca(1, 1)
        gat(5, 5)

        def step(i, carry):
            j0 = pl.multiple_of(i * 6 + 2, 2)
            for t in range(6):
                b = (2 + t) % 6
                j = j0 + t
                gatw(j, b)
                sca(j, b)
                scaw(j - 2, (b + 4) % 6)
                gat(jnp.minimum(j + 4, CH - 1), (b + 4) % 6)
            return carry

        lax.fori_loop(0, (CH - 4) // 6, step, 0)
        gatw(CH - 2, (CH - 2) % 6)
        sca(CH - 2, (CH - 2) % 6)
        gatw(CH - 1, (CH - 1) % 6)
        sca(CH - 1, (CH - 1) % 6)
        gatw(CH - 1, CH % 6)
        gatw(CH - 1, (CH + 1) % 6)
        scaw(CH - 4, (CH - 4) % 6)
        scaw(CH - 3, (CH - 3) % 6)
        scaw(CH - 2, (CH - 2) % 6)
        scaw(CH - 1, (CH - 1) % 6)

    plsc.subcore_barrier()
    for k in range(srows // K):
        off = pl.multiple_of(s * srows + k * K, 8)
        pltpu.sync_copy(acc_sh.at[pl.ds(off, K)], out_hbm.at[c, pl.ds(off, K)])


def _tc_in_body(x_ref, wc_ref, b_ref, n_ref, h0h_ref, xh_ref):
    x = x_ref[...]
    nc = n_ref[...]
    for hh in range(2):
        h = jnp.dot(x, wc_ref[hh], preferred_element_type=jnp.float32)
        h = h + b_ref[hh]
        h0h_ref[hh] = h
        xh_ref[hh] = h * nc


def _tc_in(x, wc, b2, norm1):
    return pl.pallas_call(
        _tc_in_body,
        grid=(GRID,),
        in_specs=[
            pl.BlockSpec((MB, D), lambda i: (i, 0)),
            pl.BlockSpec((2, D, HD), lambda i: (0, 0, 0)),
            pl.BlockSpec((2, HD), lambda i: (0, 0)),
            pl.BlockSpec((MB, 1), lambda i: (i, 0)),
        ],
        out_specs=[pl.BlockSpec((2, MB, HD), lambda i: (0, i, 0))] * 2,
        out_shape=[jax.ShapeDtypeStruct((2, NP, HD), jnp.float32)] * 2,
    )(x, wc, b2, norm1)


def _tc_mid_body(p_ref, n_ref, wq_ref, b_ref, h0h_ref, x1h_ref):
    nc = n_ref[...]
    y0 = p_ref[0] * nc
    y1 = p_ref[1] * nc
    for hh in range(2):
        t = (
            jnp.dot(y0, wq_ref[0, hh], preferred_element_type=jnp.float32)
            + jnp.dot(y1, wq_ref[1, hh], preferred_element_type=jnp.float32)
            + b_ref[hh]
        )
        t = jnp.maximum(t, 0.0) + h0h_ref[hh]
        x1h_ref[hh] = t * nc


def _tc_mid(p, norm1, wq, b2, h0h):
    return pl.pallas_call(
        _tc_mid_body,
        grid=(GRID,),
        in_specs=[
            pl.BlockSpec((2, MB, HD), lambda i: (0, i, 0)),
            pl.BlockSpec((MB, 1), lambda i: (i, 0)),
            pl.BlockSpec((2, 2, HD, HD), lambda i: (0, 0, 0, 0)),
            pl.BlockSpec((2, HD), lambda i: (0, 0)),
            pl.BlockSpec((2, MB, HD), lambda i: (0, i, 0)),
        ],
        out_specs=pl.BlockSpec((2, MB, HD), lambda i: (0, i, 0)),
        out_shape=jax.ShapeDtypeStruct((2, NP, HD), jnp.float32),
    )(p, norm1, wq, b2, h0h)


def _tc_out_body(p_ref, n_ref, w1_ref, b1_ref, wo_ref, bo_ref, oh_ref):
    nc = n_ref[...]
    y0 = p_ref[0] * nc
    y1 = p_ref[1] * nc
    h2 = []
    for hh in range(2):
        t = (
            jnp.dot(y0, w1_ref[0, hh], preferred_element_type=jnp.float32)
            + jnp.dot(y1, w1_ref[1, hh], preferred_element_type=jnp.float32)
            + b1_ref[hh]
        )
        h2.append(jnp.maximum(t, 0.0))
    for hh in range(2):
        oh_ref[hh] = (
            jnp.dot(h2[0], wo_ref[0, hh], preferred_element_type=jnp.float32)
            + jnp.dot(h2[1], wo_ref[1, hh], preferred_element_type=jnp.float32)
            + bo_ref[hh]
        )


def _tc_out(p, norm1, w1q, b12, woq, bo2):
    return pl.pallas_call(
        _tc_out_body,
        grid=(GRID,),
        in_specs=[
            pl.BlockSpec((2, MB, HD), lambda i: (0, i, 0)),
            pl.BlockSpec((MB, 1), lambda i: (i, 0)),
            pl.BlockSpec((2, 2, HD, HD), lambda i: (0, 0, 0, 0)),
            pl.BlockSpec((2, HD), lambda i: (0, 0)),
            pl.BlockSpec((2, 2, HD, HD), lambda i: (0, 0, 0, 0)),
            pl.BlockSpec((2, HD), lambda i: (0, 0)),
        ],
        out_specs=pl.BlockSpec((2, MB, HD), lambda i: (0, i, 0)),
        out_shape=jax.ShapeDtypeStruct((2, N, HD), jnp.float32),
    )(p, norm1, w1q, b12, woq, bo2)


def kernel(features, edge_index, W_in, b_in, W0, b0, W1, b1, W_out, b_out):
    src = edge_index[0].astype(jnp.int32)
    dst = edge_index[1].astype(jnp.int32)
    pad = EP - E
    srcp = jnp.concatenate([src, jnp.zeros((pad,), jnp.int32)])
    dstp = jnp.concatenate([dst, jnp.full((pad,), NP - 8, jnp.int32)])
    src3 = srcp.reshape(NS, 2, CH, K)
    dst3 = dstp.reshape(NS, 2, CH, K)

    degp = _sc_deg(dstp)
    deg = degp[0] + degp[1]
    norm1 = lax.rsqrt(jnp.maximum(deg, 1.0)).reshape(NP, 1)

    winc = W_in.reshape(D, 2, HD).transpose(1, 0, 2)
    w0q = W0.reshape(2, HD, 2, HD).transpose(0, 2, 1, 3)
    w1q = W1.reshape(2, HD, 2, HD).transpose(0, 2, 1, 3)
    woq = W_out.reshape(2, HD, 2, HD).transpose(0, 2, 1, 3)

    h0h, x0h = _tc_in(features, winc, b_in.reshape(2, HD), norm1)
    p = _sc_seg(x0h, src3, dst3)
    x1h = _tc_mid(p, norm1, w0q, b0.reshape(2, HD), h0h)
    p = _sc_seg(x1h, src3, dst3)
    oh = _tc_out(p, norm1, w1q, b1.reshape(2, HD), woq, b_out.reshape(2, HD))
    return jnp.concatenate([oh[0], oh[1]], axis=1)
